# Initial kernel scaffold; baseline (speedup 1.0000x reference)
import jax
import jax.numpy as jnp
from jax.experimental import pallas as pl


def kernel(byte_sequence, table_2, table_3, table_4, Wp, bp):
    raise NotImplementedError("write your pallas kernel here")


# speedup vs baseline: 3.7818x; 3.7818x over previous
import functools

import jax
import jax.numpy as jnp
from jax import lax
from jax.experimental import pallas as pl
from jax.experimental.pallas import tpu as pltpu
from jax.experimental.pallas import tpu_sc as plsc

NGRAMS = (2, 3, 4)
V = 100000
D = 64
B, S = 1024, 200
WPAD = 256

NC, NS, L = 2, 16, 16
NW = NC * NS
BPW = B // NW


def _hash_body(seq_ref, h2_ref, h3_ref, h4_ref):
    seq = seq_ref[...]
    bb = seq.shape[0]
    for n, href in zip(NGRAMS, (h2_ref, h3_ref, h4_ref)):
        w = S - n + 1
        h = jnp.zeros((bb, w), jnp.int32)
        for j in range(n):
            h = (h * 256 + lax.slice(seq, (0, j), (bb, j + w))) % V
        href[...] = jnp.pad(h, ((0, 0), (0, WPAD - w)))


def _hashes(byte_sequence):
    bb = 256
    grid = B // bb
    return pl.pallas_call(
        _hash_body,
        grid=(grid,),
        in_specs=[pl.BlockSpec((bb, S), lambda i: (i, 0))],
        out_specs=[pl.BlockSpec((bb, WPAD), lambda i: (i, 0))] * 3,
        out_shape=[jax.ShapeDtypeStruct((B, WPAD), jnp.int32)] * 3,
    )(byte_sequence)


def _proj_body(t2_ref, t3_ref, t4_ref, wp_ref, bp_ref, p2_ref, p3_ref, p4_ref):
    wp = wp_ref[...]

    def proj(t, k):
        return lax.dot_general(
            t, lax.slice(wp, (0, k * D), (D, (k + 1) * D)),
            (((1,), (1,)), ((), ())), preferred_element_type=jnp.float32)

    p2_ref[...] = proj(t2_ref[...], 0) + bp_ref[...]
    p3_ref[...] = proj(t3_ref[...], 1)
    p4_ref[...] = proj(t4_ref[...], 2)


def _projected_tables(table_2, table_3, table_4, Wp, bp):
    rb = 2000
    grid = V // rb
    tspec = pl.BlockSpec((rb, D), lambda i: (i, 0))
    return pl.pallas_call(
        _proj_body,
        grid=(grid,),
        in_specs=[tspec, tspec, tspec,
                  pl.BlockSpec((D, 3 * D), lambda i: (0, 0)),
                  pl.BlockSpec((1, D), lambda i: (0, 0))],
        out_specs=[tspec, tspec, tspec],
        out_shape=[jax.ShapeDtypeStruct((V, D), jnp.float32)] * 3,
    )(table_2, table_3, table_4, Wp, bp.reshape(1, D))


def _sc_body(p2, p3, p4, h2, h3, h4, bp_hbm, out_hbm,
             idx2, idx3, idx4, outv, buf3, buf4, biasv, sem):
    wid = lax.axis_index("s") * NC + lax.axis_index("c")
    base = wid * BPW
    pltpu.sync_copy(h2.at[pl.ds(base, BPW)], idx2)
    pltpu.sync_copy(h3.at[pl.ds(base, BPW)], idx3)
    pltpu.sync_copy(h4.at[pl.ds(base, BPW)], idx4)
    pltpu.sync_copy(bp_hbm, biasv)
    for j in range(4):
        outv[0, pl.ds(L * j, L)] = biasv[pl.ds(L * j, L)]

    def row_step(r, carry):
        copies = (
            pltpu.async_copy(p2.at[idx2.at[r, 0]], outv.at[pl.ds(1, 128)], sem),
            pltpu.async_copy(p2.at[idx2.at[r, 1, pl.ds(0, 72)]],
                             outv.at[pl.ds(129, 72)], sem),
            pltpu.async_copy(p3.at[idx3.at[r, 0]], buf3.at[pl.ds(0, 128)], sem),
            pltpu.async_copy(p3.at[idx3.at[r, 1, pl.ds(0, 72)]],
                             buf3.at[pl.ds(128, 72)], sem),
            pltpu.async_copy(p4.at[idx4.at[r, 0]], buf4.at[pl.ds(0, 128)], sem),
            pltpu.async_copy(p4.at[idx4.at[r, 1, pl.ds(0, 72)]],
                             buf4.at[pl.ds(128, 72)], sem),
        )
        for c in copies:
            c.wait()

        def add_step(s, carry2):
            for j in range(4):
                plsc.addupdate(outv.at[1 + s, pl.ds(L * j, L)],
                               buf3[s, pl.ds(L * j, L)])
            for j in range(4):
                plsc.addupdate(outv.at[2 + s, pl.ds(L * j, L)],
                               buf4[s, pl.ds(L * j, L)])
            return carry2

        lax.fori_loop(0, 197, add_step, 0, unroll=2)
        for j in range(4):
            plsc.addupdate(outv.at[198, pl.ds(L * j, L)],
                           buf3[197, pl.ds(L * j, L)])
        pltpu.sync_copy(outv.at[pl.ds(0, S)], out_hbm.at[base + r])
        return carry

    lax.fori_loop(0, BPW, row_step, 0)


def _sc_gather(P2, P3, P4, h2, h3, h4, bp):
    mesh = plsc.VectorSubcoreMesh(core_axis_name="c", subcore_axis_name="s",
                                  num_cores=NC, num_subcores=NS)
    f = pl.kernel(
        _sc_body,
        out_type=jax.ShapeDtypeStruct((B, S, D), jnp.float32),
        mesh=mesh,
        compiler_params=pltpu.CompilerParams(use_tc_tiling_on_sc=False),
        scratch_types=[
            pltpu.VMEM((BPW, 2, 128), jnp.int32),
            pltpu.VMEM((BPW, 2, 128), jnp.int32),
            pltpu.VMEM((BPW, 2, 128), jnp.int32),
            pltpu.VMEM((208, D), jnp.float32),
            pltpu.VMEM((200, D), jnp.float32),
            pltpu.VMEM((200, D), jnp.float32),
            pltpu.VMEM((D,), jnp.float32),
            pltpu.SemaphoreType.DMA,
        ],
    )
    return f(P2, P3, P4, h2, h3, h4, bp)


def kernel(byte_sequence, table_2, table_3, table_4, Wp, bp):
    h2, h3, h4 = _hashes(byte_sequence)
    P2, P3, P4 = _projected_tables(table_2, table_3, table_4, Wp, bp)
    h2 = h2.reshape(B, 2, 128)
    h3 = h3.reshape(B, 2, 128)
    h4 = h4.reshape(B, 2, 128)
    return _sc_gather(P2, P3, P4, h2, h3, h4, bp)


# speedup vs baseline: 3.9884x; 1.0546x over previous
import jax
import jax.numpy as jnp
from jax import lax
from jax.experimental import pallas as pl
from jax.experimental.pallas import tpu as pltpu
from jax.experimental.pallas import tpu_sc as plsc

NGRAMS = (2, 3, 4)
V = 100000
D = 64
B, S = 1024, 200
WPAD = 256

NC, NS, L = 2, 16, 16
NW = NC * NS
BPW = B // NW


def _hash_body(seq_ref, h2_ref, h3_ref, h4_ref):
    seq = seq_ref[...]
    bb = seq.shape[0]
    for n, href in zip(NGRAMS, (h2_ref, h3_ref, h4_ref)):
        w = S - n + 1
        h = jnp.zeros((bb, w), jnp.int32)
        for j in range(n):
            h = (h * 256 + lax.slice(seq, (0, j), (bb, j + w))) % V
        hp = jnp.pad(h, ((0, 0), (0, WPAD - w)))
        href[...] = jnp.stack(
            [lax.slice(hp, (0, 0), (bb, 128)),
             lax.slice(hp, (0, 128), (bb, 256))], axis=1).reshape(2 * bb, 128)


def _hashes(byte_sequence):
    bb = 256
    grid = B // bb
    return pl.pallas_call(
        _hash_body,
        grid=(grid,),
        in_specs=[pl.BlockSpec((bb, S), lambda i: (i, 0))],
        out_specs=[pl.BlockSpec((2 * bb, 128), lambda i: (i, 0))] * 3,
        out_shape=[jax.ShapeDtypeStruct((2 * B, 128), jnp.int32)] * 3,
    )(byte_sequence)


def _proj_body(t2_ref, t3_ref, t4_ref, wp_ref, p2_ref, p3_ref, p4_ref):
    wp = wp_ref[...]

    def proj(t, k):
        w = lax.slice(wp, (0, k * D), (D, (k + 1) * D))
        dg = lambda x: lax.dot_general(x, w, (((1,), (1,)), ((), ())),
                                       preferred_element_type=jnp.float32)
        left = dg(lax.slice(t, (0, 0), (t.shape[0], D)))
        right = dg(lax.slice(t, (0, D), (t.shape[0], 2 * D)))
        return jnp.concatenate([left, right], axis=1)

    p2_ref[...] = proj(t2_ref[...], 0)
    p3_ref[...] = proj(t3_ref[...], 1)
    p4_ref[...] = proj(t4_ref[...], 2)


def _projected_tables(t2p, t3p, t4p, Wp):
    rb = 1000
    grid = (V // 2) // rb
    tspec = pl.BlockSpec((rb, 2 * D), lambda i: (i, 0))
    return pl.pallas_call(
        _proj_body,
        grid=(grid,),
        in_specs=[tspec, tspec, tspec,
                  pl.BlockSpec((D, 3 * D), lambda i: (0, 0))],
        out_specs=[tspec, tspec, tspec],
        out_shape=[jax.ShapeDtypeStruct((V // 2, 2 * D), jnp.float32)] * 3,
    )(t2p, t3p, t4p, Wp)


def _sc_body(p2, p3, p4, h2, h3, h4, out_hbm,
             idx2, idx3, idx4, outv, buf3, buf4, sem):
    wid = lax.axis_index("s") * NC + lax.axis_index("c")
    base = wid * BPW
    pltpu.sync_copy(h2.at[pl.ds(base, BPW)], idx2)
    pltpu.sync_copy(h3.at[pl.ds(base, BPW)], idx3)
    pltpu.sync_copy(h4.at[pl.ds(base, BPW)], idx4)
    zeros = jnp.zeros((L,), jnp.float32)
    for j in range(4):
        outv[0, pl.ds(L * j, L)] = zeros

    def row_step(r, carry):
        copies = (
            pltpu.async_copy(p2.at[idx2.at[r, 0]], outv.at[pl.ds(1, 128)], sem),
            pltpu.async_copy(p2.at[idx2.at[r, 1, pl.ds(0, 72)]],
                             outv.at[pl.ds(129, 72)], sem),
            pltpu.async_copy(p3.at[idx3.at[r, 0]], buf3.at[pl.ds(0, 128)], sem),
            pltpu.async_copy(p3.at[idx3.at[r, 1, pl.ds(0, 72)]],
                             buf3.at[pl.ds(128, 72)], sem),
            pltpu.async_copy(p4.at[idx4.at[r, 0]], buf4.at[pl.ds(0, 128)], sem),
            pltpu.async_copy(p4.at[idx4.at[r, 1, pl.ds(0, 72)]],
                             buf4.at[pl.ds(128, 72)], sem),
        )
        for c in copies:
            c.wait()

        def add_step(s, carry2):
            for j in range(4):
                plsc.addupdate(outv.at[1 + s, pl.ds(L * j, L)],
                               buf3[s, pl.ds(L * j, L)])
            for j in range(4):
                plsc.addupdate(outv.at[2 + s, pl.ds(L * j, L)],
                               buf4[s, pl.ds(L * j, L)])
            return carry2

        lax.fori_loop(0, 197, add_step, 0, unroll=2)
        for j in range(4):
            plsc.addupdate(outv.at[198, pl.ds(L * j, L)],
                           buf3[197, pl.ds(L * j, L)])
        pltpu.sync_copy(outv.at[pl.ds(0, S)], out_hbm.at[base + r])
        return carry

    lax.fori_loop(0, BPW, row_step, 0)


def _sc_gather(P2, P3, P4, h2, h3, h4):
    mesh = plsc.VectorSubcoreMesh(core_axis_name="c", subcore_axis_name="s",
                                  num_cores=NC, num_subcores=NS)
    f = pl.kernel(
        _sc_body,
        out_type=jax.ShapeDtypeStruct((B, S, D), jnp.float32),
        mesh=mesh,
        compiler_params=pltpu.CompilerParams(use_tc_tiling_on_sc=False),
        scratch_types=[
            pltpu.VMEM((BPW, 2, 128), jnp.int32),
            pltpu.VMEM((BPW, 2, 128), jnp.int32),
            pltpu.VMEM((BPW, 2, 128), jnp.int32),
            pltpu.VMEM((208, D), jnp.float32),
            pltpu.VMEM((200, D), jnp.float32),
            pltpu.VMEM((200, D), jnp.float32),
            pltpu.SemaphoreType.DMA,
        ],
    )
    return f(P2, P3, P4, h2, h3, h4)


def _final_body(acc_ref, bp_ref, out_ref):
    x = acc_ref[...]
    bb = out_ref.shape[0]
    even = lax.slice(x, (0, 0), (x.shape[0], D))
    odd = lax.slice(x, (0, D), (x.shape[0], 2 * D))
    out_ref[...] = jnp.stack([even, odd], axis=1).reshape(bb, S, D) + bp_ref[...]


def _finalize(acc2, bp):
    bb = 128
    grid = B // bb
    return pl.pallas_call(
        _final_body,
        grid=(grid,),
        in_specs=[pl.BlockSpec((bb * 100, 128), lambda i: (i, 0)),
                  pl.BlockSpec((1, 1, D), lambda i: (0, 0, 0))],
        out_specs=pl.BlockSpec((bb, S, D), lambda i: (i, 0, 0)),
        out_shape=jax.ShapeDtypeStruct((B, S, D), jnp.float32),
    )(acc2, bp.reshape(1, 1, D))


def kernel(byte_sequence, table_2, table_3, table_4, Wp, bp):
    h2, h3, h4 = _hashes(byte_sequence)
    P2, P3, P4 = _projected_tables(
        table_2.reshape(V // 2, 2 * D), table_3.reshape(V // 2, 2 * D),
        table_4.reshape(V // 2, 2 * D), Wp)
    acc = _sc_gather(
        P2.reshape(V, D), P3.reshape(V, D), P4.reshape(V, D),
        h2.reshape(B, 2, 128), h3.reshape(B, 2, 128), h4.reshape(B, 2, 128))
    return _finalize(acc.reshape(B * 100, 128), bp)


# speedup vs baseline: 4.2715x; 1.0710x over previous
import jax
import jax.numpy as jnp
from jax import lax
from jax.experimental import pallas as pl
from jax.experimental.pallas import tpu as pltpu
from jax.experimental.pallas import tpu_sc as plsc

NGRAMS = (2, 3, 4)
V = 100000
D = 64
B, S = 1024, 200
WPAD = 256

NC, NS, L = 2, 16, 16
NW = NC * NS
BPW = B // NW


def _hash_body(seq_ref, h2_ref, h3_ref, h4_ref):
    seq = seq_ref[...]
    bb = seq.shape[0]
    for n, href in zip(NGRAMS, (h2_ref, h3_ref, h4_ref)):
        w = S - n + 1
        h = jnp.zeros((bb, w), jnp.int32)
        for j in range(n):
            h = (h * 256 + lax.slice(seq, (0, j), (bb, j + w))) % V
        h = jnp.where(h < V // 2, 2 * h, 2 * h - (V - 1))
        hp = jnp.pad(h, ((0, 0), (0, WPAD - w)))
        href[...] = jnp.stack(
            [lax.slice(hp, (0, 0), (bb, 128)),
             lax.slice(hp, (0, 128), (bb, 256))], axis=1).reshape(2 * bb, 128)


def _hashes(byte_sequence):
    bb = 256
    grid = B // bb
    return pl.pallas_call(
        _hash_body,
        grid=(grid,),
        in_specs=[pl.BlockSpec((bb, S), lambda i: (i, 0))],
        out_specs=[pl.BlockSpec((2 * bb, 128), lambda i: (i, 0))] * 3,
        out_shape=[jax.ShapeDtypeStruct((2 * B, 128), jnp.int32)] * 3,
    )(byte_sequence)


def _proj_body(t2a_ref, t2b_ref, t3a_ref, t3b_ref, t4a_ref, t4b_ref,
               wp_ref, p2_ref, p3_ref, p4_ref):
    wp = wp_ref[...]

    def proj(ta, tb, k):
        w = lax.slice(wp, (0, k * D), (D, (k + 1) * D))
        dg = lambda x: lax.dot_general(x, w, (((1,), (1,)), ((), ())),
                                       preferred_element_type=jnp.float32)
        return jnp.concatenate([dg(ta), dg(tb)], axis=1)

    p2_ref[...] = proj(t2a_ref[...], t2b_ref[...], 0)
    p3_ref[...] = proj(t3a_ref[...], t3b_ref[...], 1)
    p4_ref[...] = proj(t4a_ref[...], t4b_ref[...], 2)


def _projected_tables(table_2, table_3, table_4, Wp):
    rb = 1000
    grid = (V // 2) // rb
    top = pl.BlockSpec((rb, D), lambda i: (i, 0))
    bot = pl.BlockSpec((rb, D), lambda i: (i + grid, 0))
    return pl.pallas_call(
        _proj_body,
        grid=(grid,),
        in_specs=[top, bot, top, bot, top, bot,
                  pl.BlockSpec((D, 3 * D), lambda i: (0, 0))],
        out_specs=[pl.BlockSpec((rb, 2 * D), lambda i: (i, 0))] * 3,
        out_shape=[jax.ShapeDtypeStruct((V // 2, 2 * D), jnp.float32)] * 3,
    )(table_2, table_2, table_3, table_3, table_4, table_4, Wp)


def _sc_body(p2, p3, p4, h2, h3, h4, out_hbm,
             idx2, idx3, idx4, outv, buf3, buf4, sem):
    wid = lax.axis_index("s") * NC + lax.axis_index("c")
    base = wid * BPW
    pltpu.sync_copy(h2.at[pl.ds(base, BPW)], idx2)
    pltpu.sync_copy(h3.at[pl.ds(base, BPW)], idx3)
    pltpu.sync_copy(h4.at[pl.ds(base, BPW)], idx4)
    zeros = jnp.zeros((L,), jnp.float32)
    for j in range(4):
        outv[0, pl.ds(L * j, L)] = zeros

    def row_step(r, carry):
        copies = (
            pltpu.async_copy(p2.at[idx2.at[r, 0]], outv.at[pl.ds(1, 128)], sem),
            pltpu.async_copy(p2.at[idx2.at[r, 1, pl.ds(0, 72)]],
                             outv.at[pl.ds(129, 72)], sem),
            pltpu.async_copy(p3.at[idx3.at[r, 0]], buf3.at[pl.ds(0, 128)], sem),
            pltpu.async_copy(p3.at[idx3.at[r, 1, pl.ds(0, 72)]],
                             buf3.at[pl.ds(128, 72)], sem),
            pltpu.async_copy(p4.at[idx4.at[r, 0]], buf4.at[pl.ds(0, 128)], sem),
            pltpu.async_copy(p4.at[idx4.at[r, 1, pl.ds(0, 72)]],
                             buf4.at[pl.ds(128, 72)], sem),
        )
        for c in copies:
            c.wait()

        def add_step(s, carry2):
            for j in range(4):
                plsc.addupdate(outv.at[1 + s, pl.ds(L * j, L)],
                               buf3[s, pl.ds(L * j, L)])
            for j in range(4):
                plsc.addupdate(outv.at[2 + s, pl.ds(L * j, L)],
                               buf4[s, pl.ds(L * j, L)])
            return carry2

        lax.fori_loop(0, 197, add_step, 0, unroll=2)
        for j in range(4):
            plsc.addupdate(outv.at[198, pl.ds(L * j, L)],
                           buf3[197, pl.ds(L * j, L)])
        pltpu.sync_copy(outv.at[pl.ds(0, S)], out_hbm.at[base + r])
        return carry

    lax.fori_loop(0, BPW, row_step, 0)


def _sc_gather(P2, P3, P4, h2, h3, h4):
    mesh = plsc.VectorSubcoreMesh(core_axis_name="c", subcore_axis_name="s",
                                  num_cores=NC, num_subcores=NS)
    f = pl.kernel(
        _sc_body,
        out_type=jax.ShapeDtypeStruct((B, S, D), jnp.float32),
        mesh=mesh,
        compiler_params=pltpu.CompilerParams(use_tc_tiling_on_sc=False),
        scratch_types=[
            pltpu.VMEM((BPW, 2, 128), jnp.int32),
            pltpu.VMEM((BPW, 2, 128), jnp.int32),
            pltpu.VMEM((BPW, 2, 128), jnp.int32),
            pltpu.VMEM((208, D), jnp.float32),
            pltpu.VMEM((200, D), jnp.float32),
            pltpu.VMEM((200, D), jnp.float32),
            pltpu.SemaphoreType.DMA,
        ],
    )
    return f(P2, P3, P4, h2, h3, h4)


def _final_body(acc_ref, bp_ref, out_ref):
    x = acc_ref[...]
    bb = out_ref.shape[0]
    even = lax.slice(x, (0, 0), (x.shape[0], D))
    odd = lax.slice(x, (0, D), (x.shape[0], 2 * D))
    out_ref[...] = jnp.stack([even, odd], axis=1).reshape(bb, S, D) + bp_ref[...]


def _finalize(acc2, bp):
    bb = 128
    grid = B // bb
    return pl.pallas_call(
        _final_body,
        grid=(grid,),
        in_specs=[pl.BlockSpec((bb * 100, 128), lambda i: (i, 0)),
                  pl.BlockSpec((1, 1, D), lambda i: (0, 0, 0))],
        out_specs=pl.BlockSpec((bb, S, D), lambda i: (i, 0, 0)),
        out_shape=jax.ShapeDtypeStruct((B, S, D), jnp.float32),
    )(acc2, bp.reshape(1, 1, D))


def kernel(byte_sequence, table_2, table_3, table_4, Wp, bp):
    h2, h3, h4 = _hashes(byte_sequence)
    P2, P3, P4 = _projected_tables(table_2, table_3, table_4, Wp)
    acc = _sc_gather(
        P2.reshape(V, D), P3.reshape(V, D), P4.reshape(V, D),
        h2.reshape(B, 2, 128), h3.reshape(B, 2, 128), h4.reshape(B, 2, 128))
    return _finalize(acc.reshape(B * 100, 128), bp)
